# baseline (device time: 20679 ns/iter reference)
import jax
import jax.numpy as jnp
from jax import lax
from jax.experimental import pallas as pl
from jax.experimental.pallas import tpu as pltpu

CM = 64
CQ = 8
NX = CQ + 3
NY = CQ + 2
NZ = CQ + 3


def kernel(x):
    m_per, n = x.shape
    qr = m_per // 4

    def body(x_ref, out_ref, xs, xr, ys, yr, zs, zr):
        my_x = lax.axis_index("x")
        my_y = lax.axis_index("y")
        my_z = lax.axis_index("z")
        peer_x = (1 - my_x, my_y, my_z)
        peer_y = (my_x, 1 - my_y, my_z)
        peer_z = (my_x, my_y, 1 - my_z)

        my_idx = 2 * my_y + my_z
        d_y = 2 * my_y + (1 - my_z)
        d_z = 2 * (1 - my_y) + my_z
        diag = 2 * (1 - my_y) + (1 - my_z)

        own_base = my_x * m_per
        far_base = (1 - my_x) * m_per

        def rdma(rows, send_sem, recv_sem, peer):
            sl = out_ref.at[pl.ds(rows, CM), :]
            return pltpu.make_async_remote_copy(
                src_ref=sl, dst_ref=sl,
                send_sem=send_sem, recv_sem=recv_sem,
                device_id=peer, device_id_type=pl.DeviceIdType.MESH,
            )

        out_ref[pl.ds(own_base, m_per), :] = x_ref[...].astype(jnp.bfloat16)

        barrier_sem = pltpu.get_barrier_semaphore()
        for p in (peer_x, peer_y, peer_z):
            pl.semaphore_signal(
                barrier_sem, inc=1, device_id=p,
                device_id_type=pl.DeviceIdType.MESH,
            )
        pl.semaphore_wait(barrier_sem, 3)

        x_out = []
        for c in range(CQ):
            r = rdma(own_base + my_idx * qr + c * CM,
                     xs.at[c], xr.at[c], peer_x)
            r.start()
            x_out.append(r)
        for k in range(3):
            r = rdma(own_base + diag * qr + k * CM,
                     xs.at[CQ + k], xr.at[CQ + k], peer_x)
            r.start()
            x_out.append(r)

        y_out, z_out = [], []
        for c in range(CQ):
            rows = far_base + my_idx * qr + c * CM
            rdma(rows, xs.at[c], xr.at[c], peer_x).wait_recv()
            ry = rdma(rows, ys.at[c], yr.at[c], peer_y)
            ry.start()
            y_out.append(ry)
            rz = rdma(rows, zs.at[c], zr.at[c], peer_z)
            rz.start()
            z_out.append(rz)

        for j, k in enumerate((3, 4)):
            rdma(far_base + d_y * qr + k * CM, zs.at[k], zr.at[k],
                 peer_z).wait_recv()
            ry = rdma(far_base + d_y * qr + k * CM,
                      ys.at[CQ + j], yr.at[CQ + j], peer_y)
            ry.start()
            y_out.append(ry)
        for j, k in enumerate((5, 6, 7)):
            rdma(far_base + d_z * qr + k * CM, ys.at[k], yr.at[k],
                 peer_y).wait_recv()
            rz = rdma(far_base + d_z * qr + k * CM,
                      zs.at[CQ + j], zr.at[CQ + j], peer_z)
            rz.start()
            z_out.append(rz)

        for k in range(3):
            rdma(far_base + diag * qr + k * CM, xs.at[CQ + k],
                 xr.at[CQ + k], peer_x).wait_recv()
        for k in range(5):
            rdma(far_base + d_z * qr + k * CM, ys.at[k], yr.at[k],
                 peer_y).wait_recv()
        for k in (0, 1, 2, 5, 6, 7):
            rdma(far_base + d_y * qr + k * CM, zs.at[k], zr.at[k],
                 peer_z).wait_recv()
        for j, k in enumerate((3, 4)):
            rdma(far_base + diag * qr + k * CM, ys.at[CQ + j],
                 yr.at[CQ + j], peer_y).wait_recv()
        for j, k in enumerate((5, 6, 7)):
            rdma(far_base + diag * qr + k * CM, zs.at[CQ + j],
                 zr.at[CQ + j], peer_z).wait_recv()

        for r in x_out + y_out + z_out:
            r.wait_send()

    return pl.pallas_call(
        body,
        out_shape=jax.ShapeDtypeStruct((2 * m_per, n), jnp.bfloat16),
        in_specs=[pl.BlockSpec(memory_space=pltpu.VMEM)],
        out_specs=pl.BlockSpec(memory_space=pltpu.VMEM),
        scratch_shapes=[
            pltpu.SemaphoreType.DMA((NX,)),
            pltpu.SemaphoreType.DMA((NX,)),
            pltpu.SemaphoreType.DMA((NY,)),
            pltpu.SemaphoreType.DMA((NY,)),
            pltpu.SemaphoreType.DMA((NZ,)),
            pltpu.SemaphoreType.DMA((NZ,)),
        ],
        compiler_params=pltpu.CompilerParams(collective_id=0),
    )(x)


# device time: 20497 ns/iter; 1.0089x vs baseline; 1.0089x over previous
import jax
import jax.numpy as jnp
from jax import lax
from jax.experimental import pallas as pl
from jax.experimental.pallas import tpu as pltpu

CM = 64
CQ = 8
NX = CQ + 3
NY = CQ + 2
NZ = CQ + 3


def kernel(x):
    m_per, n = x.shape
    qr = m_per // 4

    def body(x_ref, out_ref, xs, xr, ys, yr, zs, zr):
        my_x = lax.axis_index("x")
        my_y = lax.axis_index("y")
        my_z = lax.axis_index("z")
        peer_x = (1 - my_x, my_y, my_z)
        peer_y = (my_x, 1 - my_y, my_z)
        peer_z = (my_x, my_y, 1 - my_z)

        my_idx = 2 * my_y + my_z
        d_y = 2 * my_y + (1 - my_z)
        d_z = 2 * (1 - my_y) + my_z
        diag = 2 * (1 - my_y) + (1 - my_z)

        own_base = my_x * m_per
        far_base = (1 - my_x) * m_per

        def rdma(rows, send_sem, recv_sem, peer):
            sl = out_ref.at[pl.ds(rows, CM), :]
            return pltpu.make_async_remote_copy(
                src_ref=sl, dst_ref=sl,
                send_sem=send_sem, recv_sem=recv_sem,
                device_id=peer, device_id_type=pl.DeviceIdType.MESH,
            )

        barrier_sem = pltpu.get_barrier_semaphore()
        for p in (peer_x, peer_y, peer_z):
            pl.semaphore_signal(
                barrier_sem, inc=1, device_id=p,
                device_id_type=pl.DeviceIdType.MESH,
            )
        pl.semaphore_wait(barrier_sem, 3)

        x_out = []
        for c in range(CQ):
            loc = my_idx * qr + c * CM
            out_ref[pl.ds(own_base + loc, CM), :] = x_ref[
                pl.ds(loc, CM), :
            ].astype(jnp.bfloat16)
            r = rdma(own_base + loc, xs.at[c], xr.at[c], peer_x)
            r.start()
            x_out.append(r)
        for k in range(3):
            loc = diag * qr + k * CM
            out_ref[pl.ds(own_base + loc, CM), :] = x_ref[
                pl.ds(loc, CM), :
            ].astype(jnp.bfloat16)
            r = rdma(own_base + loc, xs.at[CQ + k], xr.at[CQ + k], peer_x)
            r.start()
            x_out.append(r)

        for q in range(4):
            @pl.when((q != my_idx) & (q != diag))
            def _(q=q):
                out_ref[pl.ds(own_base + q * qr, qr), :] = x_ref[
                    q * qr : (q + 1) * qr, :
                ].astype(jnp.bfloat16)
        tail = qr - 3 * CM
        out_ref[pl.ds(own_base + diag * qr + 3 * CM, tail), :] = x_ref[
            pl.ds(diag * qr + 3 * CM, tail), :
        ].astype(jnp.bfloat16)

        y_out, z_out = [], []
        for c in range(CQ):
            rows = far_base + my_idx * qr + c * CM
            rdma(rows, xs.at[c], xr.at[c], peer_x).wait_recv()
            ry = rdma(rows, ys.at[c], yr.at[c], peer_y)
            ry.start()
            y_out.append(ry)
            rz = rdma(rows, zs.at[c], zr.at[c], peer_z)
            rz.start()
            z_out.append(rz)

        for j, k in enumerate((3, 4)):
            rdma(far_base + d_y * qr + k * CM, zs.at[k], zr.at[k],
                 peer_z).wait_recv()
            ry = rdma(far_base + d_y * qr + k * CM,
                      ys.at[CQ + j], yr.at[CQ + j], peer_y)
            ry.start()
            y_out.append(ry)
        for j, k in enumerate((5, 6, 7)):
            rdma(far_base + d_z * qr + k * CM, ys.at[k], yr.at[k],
                 peer_y).wait_recv()
            rz = rdma(far_base + d_z * qr + k * CM,
                      zs.at[CQ + j], zr.at[CQ + j], peer_z)
            rz.start()
            z_out.append(rz)

        for k in range(3):
            rdma(far_base + diag * qr + k * CM, xs.at[CQ + k],
                 xr.at[CQ + k], peer_x).wait_recv()
        for k in range(5):
            rdma(far_base + d_z * qr + k * CM, ys.at[k], yr.at[k],
                 peer_y).wait_recv()
        for k in (0, 1, 2, 5, 6, 7):
            rdma(far_base + d_y * qr + k * CM, zs.at[k], zr.at[k],
                 peer_z).wait_recv()
        for j, k in enumerate((3, 4)):
            rdma(far_base + diag * qr + k * CM, ys.at[CQ + j],
                 yr.at[CQ + j], peer_y).wait_recv()
        for j, k in enumerate((5, 6, 7)):
            rdma(far_base + diag * qr + k * CM, zs.at[CQ + j],
                 zr.at[CQ + j], peer_z).wait_recv()

        for r in x_out + y_out + z_out:
            r.wait_send()

    return pl.pallas_call(
        body,
        out_shape=jax.ShapeDtypeStruct((2 * m_per, n), jnp.bfloat16),
        in_specs=[pl.BlockSpec(memory_space=pltpu.VMEM)],
        out_specs=pl.BlockSpec(memory_space=pltpu.VMEM),
        scratch_shapes=[
            pltpu.SemaphoreType.DMA((NX,)),
            pltpu.SemaphoreType.DMA((NX,)),
            pltpu.SemaphoreType.DMA((NY,)),
            pltpu.SemaphoreType.DMA((NY,)),
            pltpu.SemaphoreType.DMA((NZ,)),
            pltpu.SemaphoreType.DMA((NZ,)),
        ],
        compiler_params=pltpu.CompilerParams(collective_id=0),
    )(x)
